# baseline (speedup 1.0000x reference)
import functools

import jax
import jax.numpy as jnp
from jax import lax
from jax.experimental import pallas as pl
from jax.experimental.pallas import tpu as pltpu
from jax.experimental.pallas import tpu_sc as plsc

N = 10000
N_PAD = 10240
E = 160000
NW = 32
CHUNK = 128
NCH = 40
E_PAD = NW * NCH * CHUNK
EPT = 5120
SLOW_CID = 1
FAST_EPT = 5120
SLOW_EPT = 5120
BN = 256
NB = N_PAD // BN
RPT = N_PAD // 16
ZR = 32


def _deg_kernel():
    mesh = plsc.VectorSubcoreMesh(core_axis_name="c", subcore_axis_name="s")

    @functools.partial(
        pl.kernel,
        out_type=jax.ShapeDtypeStruct((2, N_PAD, 16), jnp.float32),
        mesh=mesh,
        compiler_params=pltpu.CompilerParams(use_tc_tiling_on_sc=False),
        scratch_types=[
            pltpu.VMEM_SHARED((N_PAD, 16), jnp.float32),
            pltpu.VMEM((ZR, 16), jnp.float32),
            pltpu.VMEM((CHUNK, 16), jnp.float32),
            pltpu.VMEM((NCH, CHUNK), jnp.int32),
            pltpu.SemaphoreType.DMA,
            pltpu.SemaphoreType.DMA,
        ],
    )
    def k(dstr_hbm, out_hbm, acc, zbuf, ones, didx, ssem, zsem):
        cid = lax.axis_index("c")
        sid = lax.axis_index("s")
        wid = cid * 16 + sid
        zero16 = jnp.zeros((16,), jnp.float32)
        one16 = jnp.ones((16,), jnp.float32)

        def fill(r, _):
            zbuf[r, pl.ds(0, 16)] = zero16
            return 0
        lax.fori_loop(0, ZR, fill, 0)

        def fill1(r, _):
            ones[r, pl.ds(0, 16)] = one16
            return 0
        lax.fori_loop(0, CHUNK, fill1, 0)

        pltpu.sync_copy(dstr_hbm.at[wid], didx)
        for k2 in range(RPT // ZR):
            pltpu.async_copy(zbuf, acc.at[pl.ds(sid * RPT + k2 * ZR, ZR)], zsem)
        for k2 in range(RPT // ZR):
            pltpu.make_async_copy(zbuf, acc.at[pl.ds(sid * RPT, ZR)], zsem).wait()
        plsc.subcore_barrier()

        G = 8

        def body(g2, _):
            for b in range(G):
                pltpu.async_copy(ones, acc.at[didx.at[g2 * G + b]], ssem,
                                 add=True)
            for b in range(G):
                pltpu.make_async_copy(ones, acc.at[didx.at[0]], ssem).wait()
            return 0
        lax.fori_loop(0, NCH // G, body, 0)
        plsc.subcore_barrier()
        pltpu.sync_copy(acc.at[pl.ds(sid * RPT, RPT)],
                        out_hbm.at[cid, pl.ds(sid * RPT, RPT)])

    return k


def _agg_kernel(C):
    D = 64
    mesh = plsc.VectorSubcoreMesh(core_axis_name="c", subcore_axis_name="s")
    CH = 64
    NCHF = FAST_EPT // CH
    NCHS = SLOW_EPT // CH
    H = 2
    NBUF = 2 * H
    NGF = NCHF // NBUF
    NGS = NCHS // NBUF
    assert NCHF % NBUF == 0 and NCHS % NBUF == 0

    @functools.partial(
        pl.kernel,
        out_type=jax.ShapeDtypeStruct((2, C, N_PAD, D), jnp.float32),
        mesh=mesh,
        compiler_params=pltpu.CompilerParams(use_tc_tiling_on_sc=False),
        scratch_types=[
            pltpu.VMEM_SHARED((N_PAD, D), jnp.float32),
            pltpu.VMEM_SHARED((N_PAD, D), jnp.float32),
            pltpu.VMEM((ZR, D), jnp.float32),
            pltpu.VMEM((NCHF, CH), jnp.int32),
            pltpu.VMEM((NCHF, CH), jnp.int32),
            pltpu.VMEM((NBUF, CH, D), jnp.float32),
            pltpu.SemaphoreType.DMA,
            pltpu.SemaphoreType.DMA,
            pltpu.SemaphoreType.DMA,
            pltpu.SemaphoreType.DMA,
            pltpu.SemaphoreType.DMA,
            pltpu.SemaphoreType.DMA,
        ],
    )
    def k(g_hbm, srcr_hbm, dstr_hbm, out_hbm, tbl, acc, zbuf, sidx, didx,
          rows, gsa, gsb, ssa, ssb, zsem, tsem):
        cid = lax.axis_index("c")
        sid = lax.axis_index("s")
        ng = jnp.where(cid == SLOW_CID, NGS, NGF)
        zero16 = jnp.zeros((16,), jnp.float32)

        with jax.named_scope("agg_pre"):
            def fill(r, _):
                for cc in range(D // 16):
                    zbuf[r, pl.ds(cc * 16, 16)] = zero16
                return 0
            lax.fori_loop(0, ZR, fill, 0)
            pltpu.sync_copy(srcr_hbm.at[cid, sid], sidx)
            pltpu.sync_copy(dstr_hbm.at[cid, sid], didx)

        def gwait(b, sem):
            pltpu.make_async_copy(tbl.at[sidx.at[0]], rows.at[b], sem).wait()

        def swait(b, sem):
            pltpu.make_async_copy(rows.at[b], acc.at[didx.at[0]], sem).wait()

        for c in range(C):
            with jax.named_scope("agg_stage"):
                pltpu.async_copy(g_hbm.at[c, pl.ds(sid * RPT, RPT)],
                                 tbl.at[pl.ds(sid * RPT, RPT)], tsem)
                for k2 in range(RPT // ZR):
                    pltpu.async_copy(zbuf,
                                     acc.at[pl.ds(sid * RPT + k2 * ZR, ZR)],
                                     zsem)
                for k2 in range(RPT // ZR):
                    pltpu.make_async_copy(zbuf, acc.at[pl.ds(sid * RPT, ZR)],
                                          zsem).wait()
                pltpu.make_async_copy(g_hbm.at[0, pl.ds(sid * RPT, RPT)],
                                      tbl.at[pl.ds(sid * RPT, RPT)],
                                      tsem).wait()
                plsc.subcore_barrier()

            def group(g2, _):
                base = g2 * NBUF

                @pl.when(g2 > 0)
                def _():
                    for b in range(H):
                        swait(b, ssa)
                for b in range(H):
                    pltpu.async_copy(tbl.at[sidx.at[base + b]], rows.at[b],
                                     gsa)
                for b in range(H):
                    gwait(b, gsa)
                for b in range(H):
                    pltpu.async_copy(rows.at[b], acc.at[didx.at[base + b]],
                                     ssa, add=True)

                @pl.when(g2 > 0)
                def _():
                    for b in range(H):
                        swait(H + b, ssb)
                for b in range(H):
                    pltpu.async_copy(tbl.at[sidx.at[base + H + b]],
                                     rows.at[H + b], gsb)
                for b in range(H):
                    gwait(H + b, gsb)
                for b in range(H):
                    pltpu.async_copy(rows.at[H + b],
                                     acc.at[didx.at[base + H + b]], ssb,
                                     add=True)
                return 0

            with jax.named_scope("agg_loop"):
                lax.fori_loop(0, ng, group, 0)
                for b in range(H):
                    swait(b, ssa)
                for b in range(H):
                    swait(H + b, ssb)
                plsc.subcore_barrier()
            with jax.named_scope("agg_wb"):
                pltpu.sync_copy(acc.at[pl.ds(sid * RPT, RPT)],
                                out_hbm.at[cid, c, pl.ds(sid * RPT, RPT)])
                if c + 1 < C:
                    plsc.subcore_barrier()

    return k


def _dinv(degp):
    def body(d_ref, o_ref):
        s = d_ref[0] + d_ref[1] + 2.0
        o_ref[...] = lax.rsqrt(s[:, 0:1])

    return pl.pallas_call(
        body,
        grid=(NB,),
        in_specs=[pl.BlockSpec((2, BN, 16), lambda n: (0, n, 0))],
        out_specs=pl.BlockSpec((BN, 1), lambda n: (n, 0)),
        out_shape=jax.ShapeDtypeStruct((N_PAD, 1), jnp.float32),
    )(degp)


def _mm(x, w, bias=None, relu=False, out_scale=None, in_scale=None,
        chunk_out=False, x_chunks=1):
    K, DOUT = w.shape
    CX = x.shape[0] if x.ndim == 3 else 1
    CO = DOUT // 64 if chunk_out else 1
    grid = (NB, CO) if chunk_out else (NB, DOUT // min(DOUT, 512))
    bd = 64 if chunk_out else min(DOUT, 512)

    x_in = x
    if x.ndim == 3:
        x_spec = pl.BlockSpec((CX, BN, 64), lambda n, o: (0, n, 0))
    else:
        x_spec = pl.BlockSpec((BN, K), lambda n, o: (n, 0))

    if chunk_out:
        w_in = w.reshape(CX, K // CX, CO, 64).transpose(2, 0, 1, 3)
        w_spec = pl.BlockSpec((1, CX, K // CX, 64), lambda n, o: (o, 0, 0, 0))
        b_in = bias.reshape(CO, 1, 64) if bias is not None else None
        b_spec = pl.BlockSpec((1, 1, 64), lambda n, o: (o, 0, 0))
        out_shape = jax.ShapeDtypeStruct((CO, N_PAD, 64), jnp.float32)
        out_spec = pl.BlockSpec((1, BN, 64), lambda n, o: (o, n, 0))
    else:
        w_in = w.reshape(CX, K // CX, DOUT)
        w_spec = pl.BlockSpec((CX, K // CX, bd), lambda n, o: (0, 0, o))
        b_in = bias
        b_spec = pl.BlockSpec((1, bd), lambda n, o: (0, o))
        out_shape = jax.ShapeDtypeStruct((N_PAD, DOUT), jnp.float32)
        out_spec = pl.BlockSpec((BN, bd), lambda n, o: (n, o))

    in_specs = [x_spec, w_spec]
    args = [x_in, w_in]
    if in_scale is not None:
        in_specs.append(pl.BlockSpec((BN, 1), lambda n, o: (n, 0)))
        args.append(in_scale)
    if bias is not None:
        in_specs.append(b_spec)
        args.append(b_in)
    if out_scale is not None:
        in_specs.append(pl.BlockSpec((BN, 1), lambda n, o: (n, 0)))
        args.append(out_scale)

    def body(*refs):
        it = iter(refs)
        x_r = next(it)
        w_r = next(it)
        isc_r = next(it) if in_scale is not None else None
        b_r = next(it) if bias is not None else None
        osc_r = next(it) if out_scale is not None else None
        o_r = next(it)
        isc = isc_r[...] if isc_r is not None else None
        wv = w_r[0] if chunk_out else w_r[...]
        if x.ndim == 3:
            z = None
            for c in range(CX):
                xc = x_r[c]
                if isc is not None:
                    xc = xc * isc
                t = jnp.dot(xc, wv[c], preferred_element_type=jnp.float32)
                z = t if z is None else z + t
        else:
            xv = x_r[...]
            if isc is not None:
                xv = xv * isc
            z = jnp.dot(xv, wv[0], preferred_element_type=jnp.float32)
        if b_r is not None:
            z = z + (b_r[0] if chunk_out else b_r[...])
        if relu:
            z = jnp.maximum(z, 0.0)
        if osc_r is not None:
            z = z * osc_r[...]
        if chunk_out:
            o_r[...] = z[None]
        else:
            o_r[...] = z

    return pl.pallas_call(
        body, grid=grid, in_specs=in_specs, out_specs=out_spec,
        out_shape=out_shape)(*args)


def _combine(S, g, dinv, bias=None, relu=False, outer_scale=False, C=1):
    D = 64
    g3 = g if C > 1 or g.ndim == 3 else g.reshape(1, N_PAD, D)
    grid = (C, NB)
    in_specs = [
        pl.BlockSpec((2, 1, BN, D), lambda c, n: (0, c, n, 0)),
        pl.BlockSpec((1, BN, D), lambda c, n: (c, n, 0)),
        pl.BlockSpec((BN, 1), lambda c, n: (n, 0)),
    ]
    args = [S, g3, dinv]
    if bias is not None:
        in_specs.append(pl.BlockSpec((1, 1, D), lambda c, n: (c, 0, 0)))
        args.append(bias.reshape(C, 1, D))

    def body(*refs):
        it = iter(refs)
        s_r = next(it)
        g_r = next(it)
        dv = next(it)[...]
        b_r = next(it) if bias is not None else None
        o_r = next(it)
        s = s_r[0, 0] + s_r[1, 0] + 2.0 * g_r[0]
        base = dv * s
        if relu:
            z = jnp.maximum(base + b_r[0], 0.0)
            if outer_scale:
                z = z * dv
        elif b_r is not None:
            z = base + b_r[0]
        else:
            z = base
        o_r[...] = z[None]

    return pl.pallas_call(
        body, grid=grid, in_specs=in_specs,
        out_specs=pl.BlockSpec((1, BN, D), lambda c, n: (c, n, 0)),
        out_shape=jax.ShapeDtypeStruct((C, N_PAD, D), jnp.float32),
    )(*args)


def kernel(x, edge_index, W0, b0, W1, b1, W2, b2, W3, b3, W4, b4, W5, b5,
           W6, b6, W7, b7):
    i32 = jnp.int32
    src = edge_index[0]
    dst = edge_index[1]
    pad = E_PAD - E
    src_p = jnp.concatenate([src, jnp.zeros((pad,), i32)])
    dst_p = jnp.concatenate([dst, jnp.full((pad,), N, i32)])
    dstr = dst_p.reshape(NW, NCH, CHUNK)

    def banks(a, ch):
        nfast = 16 * FAST_EPT
        fast = a[:nfast].reshape(16, FAST_EPT)
        slow = jnp.zeros((16, FAST_EPT), i32).at[:, :SLOW_EPT].set(
            a[nfast:].reshape(16, SLOW_EPT))
        both = jnp.stack([slow, fast] if SLOW_CID == 0 else [fast, slow])
        return both.reshape(2, 16, FAST_EPT // ch, ch)

    srcr64, dstr64 = banks(src_p, 64), banks(dst_p, 64)

    x_p = jnp.zeros((N_PAD, 128), jnp.float32).at[:N].set(x)
    W7p = jnp.zeros((256, 64), jnp.float32).at[:, :40].set(W7)
    b7p = jnp.zeros((1, 64), jnp.float32).at[0, :40].set(b7)
    bias = [b.reshape(1, -1) for b in (b0, b1, b2, b3, b4, b5, b6)] + [b7p]

    degp = _deg_kernel()(dstr)
    dinv = _dinv(degp)

    agg1 = _agg_kernel(1)
    agg2 = _agg_kernel(2)

    g0 = _mm(x_p, W0, in_scale=dinv, chunk_out=True)
    S0 = agg1(g0, srcr64, dstr64)
    u1 = _combine(S0, g0, dinv, bias[0], relu=True, outer_scale=True)
    g1 = _mm(u1, W1, chunk_out=True)
    S1 = agg1(g1, srcr64, dstr64)
    u2 = _combine(S1, g1, dinv, bias[1], relu=True, outer_scale=True)
    g2 = _mm(u2, W2, chunk_out=True)
    S2 = agg1(g2, srcr64, dstr64)
    g3 = _combine(S2, g2, dinv, bias[2], relu=True, outer_scale=True)
    S3 = agg1(g3, srcr64, dstr64)
    y3 = _combine(S3, g3, dinv)
    g4 = _mm(y3, W3, bias=bias[3], relu=True, out_scale=dinv, chunk_out=True)
    S4 = agg2(g4, srcr64, dstr64)
    y4 = _combine(S4, g4, dinv, C=2)
    u5 = _mm(y4, W4, bias=bias[4], relu=True, out_scale=dinv)
    g5 = _mm(u5, W5, chunk_out=True)
    S5 = _agg_kernel(8)(g5, srcr64, dstr64)
    u6 = _combine(S5, g5, dinv, bias[5], relu=True, outer_scale=True, C=8)
    g6 = _mm(u6, W6, chunk_out=True)
    S6 = _agg_kernel(4)(g6, srcr64, dstr64)
    u7 = _combine(S6, g6, dinv, bias[6], relu=True, outer_scale=True, C=4)
    g7 = _mm(u7, W7p, chunk_out=True)
    S7 = agg1(g7, srcr64, dstr64)
    outp = _combine(S7, g7, dinv, bias[7])
    return outp[0][:N, :40]


# speedup vs baseline: 1.2775x; 1.2775x over previous
import functools

import jax
import jax.numpy as jnp
from jax import lax
from jax.experimental import pallas as pl
from jax.experimental.pallas import tpu as pltpu
from jax.experimental.pallas import tpu_sc as plsc

N = 10000
N_PAD = 10240
E = 160000
NW = 32
CHUNK = 128
NCH = 40
E_PAD = NW * NCH * CHUNK
EPT = 5120
SLOW_CID = 1
FAST_EPT = 5120
SLOW_EPT = 5120
BN = 512
NB = N_PAD // BN
RPT = N_PAD // 16
ZR = 32


def _deg_kernel():
    mesh = plsc.VectorSubcoreMesh(core_axis_name="c", subcore_axis_name="s")

    @functools.partial(
        pl.kernel,
        out_type=jax.ShapeDtypeStruct((2, N_PAD, 16), jnp.float32),
        mesh=mesh,
        compiler_params=pltpu.CompilerParams(use_tc_tiling_on_sc=False),
        scratch_types=[
            pltpu.VMEM_SHARED((N_PAD, 16), jnp.float32),
            pltpu.VMEM((ZR, 16), jnp.float32),
            pltpu.VMEM((CHUNK, 16), jnp.float32),
            pltpu.VMEM((NCH, CHUNK), jnp.int32),
            pltpu.SemaphoreType.DMA,
            pltpu.SemaphoreType.DMA,
        ],
    )
    def k(dstr_hbm, out_hbm, acc, zbuf, ones, didx, ssem, zsem):
        cid = lax.axis_index("c")
        sid = lax.axis_index("s")
        wid = cid * 16 + sid
        zero16 = jnp.zeros((16,), jnp.float32)
        one16 = jnp.ones((16,), jnp.float32)

        def fill(r, _):
            zbuf[r, pl.ds(0, 16)] = zero16
            return 0
        lax.fori_loop(0, ZR, fill, 0)

        def fill1(r, _):
            ones[r, pl.ds(0, 16)] = one16
            return 0
        lax.fori_loop(0, CHUNK, fill1, 0)

        pltpu.sync_copy(dstr_hbm.at[wid], didx)
        for k2 in range(RPT // ZR):
            pltpu.async_copy(zbuf, acc.at[pl.ds(sid * RPT + k2 * ZR, ZR)], zsem)
        for k2 in range(RPT // ZR):
            pltpu.make_async_copy(zbuf, acc.at[pl.ds(sid * RPT, ZR)], zsem).wait()
        plsc.subcore_barrier()

        G = 8

        def body(g2, _):
            for b in range(G):
                pltpu.async_copy(ones, acc.at[didx.at[g2 * G + b]], ssem,
                                 add=True)
            for b in range(G):
                pltpu.make_async_copy(ones, acc.at[didx.at[0]], ssem).wait()
            return 0
        lax.fori_loop(0, NCH // G, body, 0)
        plsc.subcore_barrier()
        pltpu.sync_copy(acc.at[pl.ds(sid * RPT, RPT)],
                        out_hbm.at[cid, pl.ds(sid * RPT, RPT)])

    return k


def _agg_kernel(C):
    D = 64
    mesh = plsc.VectorSubcoreMesh(core_axis_name="c", subcore_axis_name="s")
    CH = 64
    NCHF = FAST_EPT // CH
    NCHS = SLOW_EPT // CH
    H = 4
    NBUF = 2 * H
    NGF = NCHF // NBUF
    NGS = NCHS // NBUF
    assert NCHF % NBUF == 0 and NCHS % NBUF == 0

    @functools.partial(
        pl.kernel,
        out_type=jax.ShapeDtypeStruct((2, C, N_PAD, D), jnp.float32),
        mesh=mesh,
        compiler_params=pltpu.CompilerParams(use_tc_tiling_on_sc=False),
        scratch_types=[
            pltpu.VMEM_SHARED((N_PAD, D), jnp.float32),
            pltpu.VMEM_SHARED((N_PAD, D), jnp.float32),
            pltpu.VMEM((ZR, D), jnp.float32),
            pltpu.VMEM((NCHF, CH), jnp.int32),
            pltpu.VMEM((NCHF, CH), jnp.int32),
            pltpu.VMEM((NBUF, CH, D), jnp.float32),
            pltpu.SemaphoreType.DMA,
            pltpu.SemaphoreType.DMA,
            pltpu.SemaphoreType.DMA,
            pltpu.SemaphoreType.DMA,
            pltpu.SemaphoreType.DMA,
            pltpu.SemaphoreType.DMA,
        ],
    )
    def k(g_hbm, srcr_hbm, dstr_hbm, out_hbm, tbl, acc, zbuf, sidx, didx,
          rows, gsa, gsb, ssa, ssb, zsem, tsem):
        cid = lax.axis_index("c")
        sid = lax.axis_index("s")
        ng = jnp.where(cid == SLOW_CID, NGS, NGF)
        zero16 = jnp.zeros((16,), jnp.float32)

        with jax.named_scope("agg_pre"):
            def fill(r, _):
                for cc in range(D // 16):
                    zbuf[r, pl.ds(cc * 16, 16)] = zero16
                return 0
            lax.fori_loop(0, ZR, fill, 0)
            pltpu.sync_copy(srcr_hbm.at[cid, sid], sidx)
            pltpu.sync_copy(dstr_hbm.at[cid, sid], didx)

        def gwait(b, sem):
            pltpu.make_async_copy(tbl.at[sidx.at[0]], rows.at[b], sem).wait()

        def swait(b, sem):
            pltpu.make_async_copy(rows.at[b], acc.at[didx.at[0]], sem).wait()

        for c in range(C):
            with jax.named_scope("agg_stage"):
                pltpu.async_copy(g_hbm.at[c, pl.ds(sid * RPT, RPT)],
                                 tbl.at[pl.ds(sid * RPT, RPT)], tsem)
                for k2 in range(RPT // ZR):
                    pltpu.async_copy(zbuf,
                                     acc.at[pl.ds(sid * RPT + k2 * ZR, ZR)],
                                     zsem)
                for k2 in range(RPT // ZR):
                    pltpu.make_async_copy(zbuf, acc.at[pl.ds(sid * RPT, ZR)],
                                          zsem).wait()
                pltpu.make_async_copy(g_hbm.at[0, pl.ds(sid * RPT, RPT)],
                                      tbl.at[pl.ds(sid * RPT, RPT)],
                                      tsem).wait()
                plsc.subcore_barrier()

            def group(g2, _):
                base = g2 * NBUF

                @pl.when(g2 > 0)
                def _():
                    for b in range(H):
                        swait(b, ssa)
                for b in range(H):
                    pltpu.async_copy(tbl.at[sidx.at[base + b]], rows.at[b],
                                     gsa)
                for b in range(H):
                    gwait(b, gsa)
                for b in range(H):
                    pltpu.async_copy(rows.at[b], acc.at[didx.at[base + b]],
                                     ssa, add=True)

                @pl.when(g2 > 0)
                def _():
                    for b in range(H):
                        swait(H + b, ssb)
                for b in range(H):
                    pltpu.async_copy(tbl.at[sidx.at[base + H + b]],
                                     rows.at[H + b], gsb)
                for b in range(H):
                    gwait(H + b, gsb)
                for b in range(H):
                    pltpu.async_copy(rows.at[H + b],
                                     acc.at[didx.at[base + H + b]], ssb,
                                     add=True)
                return 0

            with jax.named_scope("agg_loop"):
                lax.fori_loop(0, ng, group, 0)
                for b in range(H):
                    swait(b, ssa)
                for b in range(H):
                    swait(H + b, ssb)
                plsc.subcore_barrier()
            with jax.named_scope("agg_wb"):
                pltpu.sync_copy(acc.at[pl.ds(sid * RPT, RPT)],
                                out_hbm.at[cid, c, pl.ds(sid * RPT, RPT)])
                if c + 1 < C:
                    plsc.subcore_barrier()

    return k


def _dinv(degp):
    def body(d_ref, o_ref):
        s = d_ref[0] + d_ref[1] + 2.0
        o_ref[...] = lax.rsqrt(s[:, 0:1])

    return pl.pallas_call(
        body,
        grid=(NB,),
        in_specs=[pl.BlockSpec((2, BN, 16), lambda n: (0, n, 0))],
        out_specs=pl.BlockSpec((BN, 1), lambda n: (n, 0)),
        out_shape=jax.ShapeDtypeStruct((N_PAD, 1), jnp.float32),
    )(degp)


def _mm(x, w, bias=None, relu=False, out_scale=None, in_scale=None,
        chunk_out=False, x_chunks=1):
    K, DOUT = w.shape
    CX = x.shape[0] if x.ndim == 3 else 1
    CO = DOUT // 64 if chunk_out else 1
    grid = (NB, CO) if chunk_out else (NB, DOUT // min(DOUT, 512))
    bd = 64 if chunk_out else min(DOUT, 512)

    x_in = x
    if x.ndim == 3:
        x_spec = pl.BlockSpec((CX, BN, 64), lambda n, o: (0, n, 0))
    else:
        x_spec = pl.BlockSpec((BN, K), lambda n, o: (n, 0))

    if chunk_out:
        w_in = w.reshape(CX, K // CX, CO, 64).transpose(2, 0, 1, 3)
        w_spec = pl.BlockSpec((1, CX, K // CX, 64), lambda n, o: (o, 0, 0, 0))
        b_in = bias.reshape(CO, 1, 64) if bias is not None else None
        b_spec = pl.BlockSpec((1, 1, 64), lambda n, o: (o, 0, 0))
        out_shape = jax.ShapeDtypeStruct((CO, N_PAD, 64), jnp.float32)
        out_spec = pl.BlockSpec((1, BN, 64), lambda n, o: (o, n, 0))
    else:
        w_in = w.reshape(CX, K // CX, DOUT)
        w_spec = pl.BlockSpec((CX, K // CX, bd), lambda n, o: (0, 0, o))
        b_in = bias
        b_spec = pl.BlockSpec((1, bd), lambda n, o: (0, o))
        out_shape = jax.ShapeDtypeStruct((N_PAD, DOUT), jnp.float32)
        out_spec = pl.BlockSpec((BN, bd), lambda n, o: (n, o))

    in_specs = [x_spec, w_spec]
    args = [x_in, w_in]
    if in_scale is not None:
        in_specs.append(pl.BlockSpec((BN, 1), lambda n, o: (n, 0)))
        args.append(in_scale)
    if bias is not None:
        in_specs.append(b_spec)
        args.append(b_in)
    if out_scale is not None:
        in_specs.append(pl.BlockSpec((BN, 1), lambda n, o: (n, 0)))
        args.append(out_scale)

    def body(*refs):
        it = iter(refs)
        x_r = next(it)
        w_r = next(it)
        isc_r = next(it) if in_scale is not None else None
        b_r = next(it) if bias is not None else None
        osc_r = next(it) if out_scale is not None else None
        o_r = next(it)
        isc = isc_r[...] if isc_r is not None else None
        wv = w_r[0] if chunk_out else w_r[...]
        if x.ndim == 3:
            z = None
            for c in range(CX):
                xc = x_r[c]
                if isc is not None:
                    xc = xc * isc
                t = jnp.dot(xc, wv[c], preferred_element_type=jnp.float32)
                z = t if z is None else z + t
        else:
            xv = x_r[...]
            if isc is not None:
                xv = xv * isc
            z = jnp.dot(xv, wv[0], preferred_element_type=jnp.float32)
        if b_r is not None:
            z = z + (b_r[0] if chunk_out else b_r[...])
        if relu:
            z = jnp.maximum(z, 0.0)
        if osc_r is not None:
            z = z * osc_r[...]
        if chunk_out:
            o_r[...] = z[None]
        else:
            o_r[...] = z

    return pl.pallas_call(
        body, grid=grid, in_specs=in_specs, out_specs=out_spec,
        out_shape=out_shape)(*args)


def _combine(S, g, dinv, bias=None, relu=False, outer_scale=False, C=1):
    D = 64
    g3 = g if C > 1 or g.ndim == 3 else g.reshape(1, N_PAD, D)
    grid = (C, NB)
    in_specs = [
        pl.BlockSpec((2, 1, BN, D), lambda c, n: (0, c, n, 0)),
        pl.BlockSpec((1, BN, D), lambda c, n: (c, n, 0)),
        pl.BlockSpec((BN, 1), lambda c, n: (n, 0)),
    ]
    args = [S, g3, dinv]
    if bias is not None:
        in_specs.append(pl.BlockSpec((1, 1, D), lambda c, n: (c, 0, 0)))
        args.append(bias.reshape(C, 1, D))

    def body(*refs):
        it = iter(refs)
        s_r = next(it)
        g_r = next(it)
        dv = next(it)[...]
        b_r = next(it) if bias is not None else None
        o_r = next(it)
        s = s_r[0, 0] + s_r[1, 0] + 2.0 * g_r[0]
        base = dv * s
        if relu:
            z = jnp.maximum(base + b_r[0], 0.0)
            if outer_scale:
                z = z * dv
        elif b_r is not None:
            z = base + b_r[0]
        else:
            z = base
        o_r[...] = z[None]

    return pl.pallas_call(
        body, grid=grid, in_specs=in_specs,
        out_specs=pl.BlockSpec((1, BN, D), lambda c, n: (c, n, 0)),
        out_shape=jax.ShapeDtypeStruct((C, N_PAD, D), jnp.float32),
    )(*args)


def kernel(x, edge_index, W0, b0, W1, b1, W2, b2, W3, b3, W4, b4, W5, b5,
           W6, b6, W7, b7):
    i32 = jnp.int32
    src = edge_index[0]
    dst = edge_index[1]
    pad = E_PAD - E
    src_p = jnp.concatenate([src, jnp.zeros((pad,), i32)])
    dst_p = jnp.concatenate([dst, jnp.full((pad,), N, i32)])
    dstr = dst_p.reshape(NW, NCH, CHUNK)

    def banks(a, ch):
        nfast = 16 * FAST_EPT
        fast = a[:nfast].reshape(16, FAST_EPT)
        slow = jnp.zeros((16, FAST_EPT), i32).at[:, :SLOW_EPT].set(
            a[nfast:].reshape(16, SLOW_EPT))
        both = jnp.stack([slow, fast] if SLOW_CID == 0 else [fast, slow])
        return both.reshape(2, 16, FAST_EPT // ch, ch)

    srcr64, dstr64 = banks(src_p, 64), banks(dst_p, 64)

    x_p = jnp.zeros((N_PAD, 128), jnp.float32).at[:N].set(x)
    W7p = jnp.zeros((256, 64), jnp.float32).at[:, :40].set(W7)
    b7p = jnp.zeros((1, 64), jnp.float32).at[0, :40].set(b7)
    bias = [b.reshape(1, -1) for b in (b0, b1, b2, b3, b4, b5, b6)] + [b7p]

    degp = _deg_kernel()(dstr)
    dinv = _dinv(degp)

    agg1 = _agg_kernel(1)
    agg2 = _agg_kernel(2)

    g0 = _mm(x_p, W0, in_scale=dinv, chunk_out=True)
    S0 = agg1(g0, srcr64, dstr64)
    u1 = _combine(S0, g0, dinv, bias[0], relu=True, outer_scale=True)
    g1 = _mm(u1, W1, chunk_out=True)
    S1 = agg1(g1, srcr64, dstr64)
    u2 = _combine(S1, g1, dinv, bias[1], relu=True, outer_scale=True)
    g2 = _mm(u2, W2, chunk_out=True)
    S2 = agg1(g2, srcr64, dstr64)
    g3 = _combine(S2, g2, dinv, bias[2], relu=True, outer_scale=True)
    S3 = agg1(g3, srcr64, dstr64)
    y3 = _combine(S3, g3, dinv)
    g4 = _mm(y3, W3, bias=bias[3], relu=True, out_scale=dinv, chunk_out=True)
    S4 = agg2(g4, srcr64, dstr64)
    y4 = _combine(S4, g4, dinv, C=2)
    u5 = _mm(y4, W4, bias=bias[4], relu=True, out_scale=dinv)
    g5 = _mm(u5, W5, chunk_out=True)
    S5 = _agg_kernel(8)(g5, srcr64, dstr64)
    u6 = _combine(S5, g5, dinv, bias[5], relu=True, outer_scale=True, C=8)
    g6 = _mm(u6, W6, chunk_out=True)
    S6 = _agg_kernel(4)(g6, srcr64, dstr64)
    u7 = _combine(S6, g6, dinv, bias[6], relu=True, outer_scale=True, C=4)
    g7 = _mm(u7, W7p, chunk_out=True)
    S7 = agg1(g7, srcr64, dstr64)
    outp = _combine(S7, g7, dinv, bias[7])
    return outp[0][:N, :40]
